# Initial kernel scaffold; baseline (speedup 1.0000x reference)
import jax
import jax.numpy as jnp
from jax.experimental import pallas as pl


def kernel(Z, R, atom_index12, emb, Win_w, Win_b, Wf1_w, Wf1_b, Wf2_w, Wf2_b, Wo1_w, Wo1_b, Wo2_w, Wo2_b):
    raise NotImplementedError("write your pallas kernel here")


# speedup vs baseline: 1.6678x; 1.6678x over previous
import functools

import jax
import jax.numpy as jnp
import numpy as np
from jax import lax
from jax.experimental import pallas as pl
from jax.experimental.pallas import tpu as pltpu
from jax.experimental.pallas import tpu_sc as plsc

N_ATOMS = 10000
N_EDGES = 320000
N_BASIS = 128
N_FILTERS = 128
N_INTER = 3
N_RBF = 20
CUTOFF = 5.0

_NC = 2
_NS = 16
_NW = _NC * _NS
_CH = 128
_NCHUNK = 79
_EPW = _CH * _NCHUNK
_EPAD = _EPW * _NW
_HALF = 5056
_ACC = 5120
_RPT = _ACC // _NS


def _ssp(x):
    return jax.nn.softplus(x) - jnp.log(2.0)


def _edge_body(h_hbm, wij_hbm, idxi_hbm, idxj_hbm, out_hbm,
               idxa_v, idxb_v, idxj_v, xj_v, wij_v, s_sh, sem):
    cid = lax.axis_index("c")
    sid = lax.axis_index("s")
    wid = cid * _NS + sid
    r0 = sid * _RPT

    pltpu.sync_copy(idxi_hbm.at[wid], idxa_v)
    pltpu.sync_copy(idxj_hbm.at[pl.ds(wid * _EPW, _EPW)], idxj_v)
    half = jnp.full((16,), _HALF, jnp.int32)

    def remap_row(j, carry):
        for k in range(8):
            sl = pl.ds(k * 16, 16)
            v = idxa_v[j, sl]
            in_a = v < half
            idxb_v[j, sl] = jnp.where(in_a, half, v - half)
            idxa_v[j, sl] = jnp.where(in_a, v, half)
        return carry

    lax.fori_loop(0, _NCHUNK, remap_row, 0)

    zf = jnp.zeros((16,), jnp.float32)

    def zero_row(r, carry):
        for k in range(8):
            xj_v[r, pl.ds(k * 16, 16)] = zf
        return carry

    for p in range(2):
        lax.fori_loop(0, _CH, zero_row, 0)
        pltpu.sync_copy(xj_v, s_sh.at[pl.ds(r0, _CH)])
        pltpu.sync_copy(xj_v, s_sh.at[pl.ds(r0 + _CH, _CH)])
        pltpu.sync_copy(xj_v.at[pl.ds(0, 64)], s_sh.at[pl.ds(r0 + 256, 64)])
        plsc.subcore_barrier()

        idx_ref = idxa_v if p == 0 else idxb_v

        def chunk(j, carry):
            pltpu.async_copy(h_hbm.at[idxj_v.at[pl.ds(j * _CH, _CH)]], xj_v,
                             sem).wait()
            pltpu.sync_copy(wij_hbm.at[wid * _NCHUNK + j], wij_v)

            def mul_row(r, c2):
                for k in range(8):
                    sl = pl.ds(k * 16, 16)
                    xj_v[r, sl] = xj_v[r, sl] * wij_v[r, sl]
                return c2

            lax.fori_loop(0, _CH, mul_row, 0)
            pltpu.sync_copy(xj_v, s_sh.at[idx_ref.at[j]], add=True)
            return carry

        lax.fori_loop(0, _NCHUNK, chunk, 0)

        plsc.subcore_barrier()
        pltpu.sync_copy(s_sh.at[pl.ds(r0, _RPT)],
                        out_hbm.at[cid, p, pl.ds(r0, _RPT)])
        plsc.subcore_barrier()


_edge_call = functools.partial(
    pl.kernel,
    out_type=jax.ShapeDtypeStruct((_NC, 2, _ACC, N_FILTERS), jnp.float32),
    mesh=plsc.VectorSubcoreMesh(core_axis_name="c", subcore_axis_name="s"),
    scratch_types=[
        pltpu.VMEM((_NCHUNK, _CH), jnp.int32),
        pltpu.VMEM((_NCHUNK, _CH), jnp.int32),
        pltpu.VMEM((_EPW,), jnp.int32),
        pltpu.VMEM((_CH, N_FILTERS), jnp.float32),
        pltpu.VMEM((_CH, N_FILTERS), jnp.float32),
        pltpu.VMEM_SHARED((_ACC, N_FILTERS), jnp.float32),
        pltpu.SemaphoreType.DMA,
    ],
)(_edge_body)


def _edge_aggregate(h, wij3d, idxi3d, idxj):
    parts = _edge_call(h, wij3d, idxi3d, idxj)
    s2 = parts[0] + parts[1]
    return jnp.concatenate([s2[0, :_HALF], s2[1, :N_ATOMS - _HALF]])


def kernel(Z, R, atom_index12, emb, Win_w, Win_b, Wf1_w, Wf1_b, Wf2_w, Wf2_b,
           Wo1_w, Wo1_b, Wo2_w, Wo2_b):
    npad = _EPAD - N_EDGES
    idx_i = atom_index12[0].astype(jnp.int32)
    idx_j = atom_index12[1].astype(jnp.int32)
    idx_i = jnp.concatenate([idx_i, jnp.zeros((npad,), jnp.int32)])
    idx_j = jnp.concatenate([idx_j, jnp.zeros((npad,), jnp.int32)])
    idxi3d = idx_i.reshape(_NW, _NCHUNK, _CH)

    vec = R[idx_i[:N_EDGES]] - R[idx_j[:N_EDGES]]
    d = jnp.sqrt(jnp.sum(vec * vec, axis=-1) + 1e-12)
    d = jnp.concatenate([d, jnp.full((npad,), 2.0 * CUTOFF, jnp.float32)])
    offsets = jnp.linspace(0.0, CUTOFF, N_RBF)
    coeff = -0.5 / (offsets[1] - offsets[0]) ** 2
    f_ij = jnp.exp(coeff * (d[:, None] - offsets[None, :]) ** 2)
    rcut = 0.5 * (jnp.cos(d * jnp.pi / CUTOFF) + 1.0) * (d < CUTOFF)

    x = jnp.take(emb, Z, axis=0)
    for l in range(N_INTER):
        h = x @ Win_w[l] + Win_b[l]
        wij = _ssp(f_ij @ Wf1_w[l] + Wf1_b[l]) @ Wf2_w[l] + Wf2_b[l]
        wij = wij * rcut[:, None]
        s = _edge_aggregate(h, wij.reshape(-1, _CH, N_FILTERS), idxi3d, idx_j)
        v = _ssp(s @ Wo1_w[l] + Wo1_b[l]) @ Wo2_w[l] + Wo2_b[l]
        x = x + v
    return x
